# Initial kernel scaffold; baseline (speedup 1.0000x reference)
import jax
import jax.numpy as jnp
from jax.experimental import pallas as pl


def kernel(x, pe):
    raise NotImplementedError("write your pallas kernel here")


# speedup vs baseline: 1.6826x; 1.6826x over previous
import jax
import jax.numpy as jnp
from jax.experimental import pallas as pl


_BS = 1024


def _add_body(x_ref, pe_ref, out_ref):
    out_ref[...] = x_ref[...] + pe_ref[...][None]


def kernel(x, pe):
    B, S, D = x.shape
    n_s = S // _BS
    return pl.pallas_call(
        _add_body,
        grid=(n_s, B),
        in_specs=[
            pl.BlockSpec((1, _BS, D), lambda i_s, i_b: (i_b, i_s, 0)),
            pl.BlockSpec((_BS, D), lambda i_s, i_b: (i_s, 0)),
        ],
        out_specs=pl.BlockSpec((1, _BS, D), lambda i_s, i_b: (i_b, i_s, 0)),
        out_shape=jax.ShapeDtypeStruct((B, S, D), x.dtype),
    )(x, pe)


# speedup vs baseline: 1.7953x; 1.0670x over previous
import jax
import jax.numpy as jnp
from jax.experimental import pallas as pl


_BS = 2048


def _add_body(x_ref, pe_ref, out_ref):
    out_ref[...] = x_ref[...] + pe_ref[...][None]


def kernel(x, pe):
    B, S, D = x.shape
    n_s = S // _BS
    return pl.pallas_call(
        _add_body,
        grid=(n_s, B),
        in_specs=[
            pl.BlockSpec((1, _BS, D), lambda i_s, i_b: (i_b, i_s, 0)),
            pl.BlockSpec((_BS, D), lambda i_s, i_b: (i_s, 0)),
        ],
        out_specs=pl.BlockSpec((1, _BS, D), lambda i_s, i_b: (i_b, i_s, 0)),
        out_shape=jax.ShapeDtypeStruct((B, S, D), x.dtype),
    )(x, pe)
